# Initial kernel scaffold; baseline (speedup 1.0000x reference)
import jax
import jax.numpy as jnp
from jax.experimental import pallas as pl


def kernel(hidden_states, pooled_output, W_p1, b_p1, W_p2, b_p2, ln_g, ln_b, Wl1, bl1, Wr1, br1, att1, bias1, Wl2, bl2, Wr2, br2, att2, bias2, W_lin1, b_lin1, W_lin2, b_lin2, W_fin, b_fin):
    raise NotImplementedError("write your pallas kernel here")


# speedup vs baseline: 159.1356x; 159.1356x over previous
import functools

import jax
import jax.numpy as jnp
from jax.experimental import pallas as pl

B = 8
S = 2048
DIN = 768
DH = 512
H = 4
C = 64
HC = 256
FT = 512
N = S + 1


def _gelu(x):
    return 0.5 * x * (1.0 + jax.lax.erf(x * 0.7071067811865476))


def _lrelu(x):
    return jnp.where(x >= 0, x, 0.2 * x)


def _graph_head_kernel(
    hs_ref, pooled_ref,
    wp1_ref, bp1_ref, wp2_ref, bp2_ref, lng_ref, lnb_ref,
    wl1_ref, bl1_ref, wr1_ref, br1_ref, p1_ref, bias1_ref,
    wl2_ref, bl2_ref, wr2_ref, br2_ref, p2_ref, bias2_ref,
    wlin1_ref, blin1_ref, wlin2_ref, blin2_ref, wfin_ref, bfin_ref,
    q_ref,
    out1_ref, out2_ref,
):
    b = pl.program_id(0)
    f32 = jnp.float32

    hs = hs_ref[0]
    p = pooled_ref[pl.ds(b, 1), :]

    h = _gelu(jnp.dot(hs, wp1_ref[...], preferred_element_type=f32) + bp1_ref[...])
    h = _gelu(jnp.dot(h, wp2_ref[...], preferred_element_type=f32) + bp2_ref[...])
    mu = jnp.mean(h, axis=-1, keepdims=True)
    var = jnp.mean((h - mu) ** 2, axis=-1, keepdims=True)
    h = (h - mu) * jax.lax.rsqrt(var + 1e-5) * lng_ref[...] + lnb_ref[...]

    q = q_ref[...]

    def gat(x_leaf, x_hub, wl_ref, bl_ref, wr_ref, br_ref, pm_ref, bias_ref):
        xl = jnp.dot(x_leaf, wl_ref[...], preferred_element_type=f32) + bl_ref[...]
        xr = jnp.dot(x_leaf, wr_ref[...], preferred_element_type=f32) + br_ref[...]
        xl0 = jnp.dot(x_hub, wl_ref[...], preferred_element_type=f32) + bl_ref[...]
        xr0 = jnp.dot(x_hub, wr_ref[...], preferred_element_type=f32) + br_ref[...]
        pm = pm_ref[...]

        e_self = jnp.dot(_lrelu(xl + xr), pm, preferred_element_type=f32)
        e_hub = jnp.dot(_lrelu(xl0 + xr), pm, preferred_element_type=f32)
        mx = jnp.maximum(e_self, e_hub)
        es = jnp.exp(e_self - mx)
        eh = jnp.exp(e_hub - mx)
        inv = 1.0 / (es + eh + 1e-16)
        a_self = jnp.dot(es * inv, q, preferred_element_type=f32)
        a_hub = jnp.dot(eh * inv, q, preferred_element_type=f32)
        y_leaf = a_self * xl + a_hub * xl0

        s_leaf = jnp.dot(_lrelu(xl + xr0), pm, preferred_element_type=f32)
        s_self = jnp.dot(_lrelu(xl0 + xr0), pm, preferred_element_type=f32)
        mxh = jnp.maximum(jnp.max(s_leaf, axis=0, keepdims=True), s_self)
        exl = jnp.exp(s_leaf - mxh)
        exs = jnp.exp(s_self - mxh)
        denom = jnp.sum(exl, axis=0, keepdims=True) + exs
        wts = jnp.dot(exl, q, preferred_element_type=f32)
        ones = jnp.ones((1, S), dtype=f32)
        num = jnp.dot(ones, wts * xl, preferred_element_type=f32)
        num = num + jnp.dot(exs, q, preferred_element_type=f32) * xl0
        y_hub = num * (1.0 / (jnp.dot(denom, q, preferred_element_type=f32) + 1e-16))

        y_leaf = _gelu(y_leaf + bias_ref[...])
        y_hub = _gelu(y_hub + bias_ref[...])
        return y_leaf, y_hub

    y1, y1h = gat(h, p, wl1_ref, bl1_ref, wr1_ref, br1_ref, p1_ref, bias1_ref)
    y2, y2h = gat(y1, y1h, wl2_ref, bl2_ref, wr2_ref, br2_ref, p2_ref, bias2_ref)

    ones = jnp.ones((1, S), dtype=f32)
    total = jnp.dot(ones, y2, preferred_element_type=f32) + y2h
    out2_ref[pl.ds(b, 1), :] = total * (1.0 / N)

    g = _gelu(jnp.dot(y2h, wlin1_ref[...], preferred_element_type=f32) + blin1_ref[...])
    g = _gelu(jnp.dot(g, wlin2_ref[...], preferred_element_type=f32) + blin2_ref[...])
    go = jnp.dot(g, wfin_ref[...], preferred_element_type=f32) + bfin_ref[...]
    out1_ref[pl.ds(b, 1), :] = go + p


@jax.jit
def kernel(hidden_states, pooled_output, W_p1, b_p1, W_p2, b_p2, ln_g, ln_b,
           Wl1, bl1, Wr1, br1, att1, bias1, Wl2, bl2, Wr2, br2, att2, bias2,
           W_lin1, b_lin1, W_lin2, b_lin2, W_fin, b_fin):
    f32 = jnp.float32
    hs3 = hidden_states[0]

    blk = (jnp.arange(HC)[:, None] // C == jnp.arange(H)[None, :]).astype(f32)
    P1 = att1.reshape(-1)[:, None] * blk
    P2 = att2.reshape(-1)[:, None] * blk
    Q = blk.T

    row = lambda v: v.reshape(1, -1)

    grid_spec = pl.GridSpec(
        grid=(B,),
        in_specs=[
            pl.BlockSpec((1, S, DIN), lambda b: (b, 0, 0)),
            pl.BlockSpec(pooled_output.shape, lambda b: (0, 0)),
            pl.BlockSpec(W_p1.shape, lambda b: (0, 0)),
            pl.BlockSpec((1, DH), lambda b: (0, 0)),
            pl.BlockSpec(W_p2.shape, lambda b: (0, 0)),
            pl.BlockSpec((1, DH), lambda b: (0, 0)),
            pl.BlockSpec((1, DH), lambda b: (0, 0)),
            pl.BlockSpec((1, DH), lambda b: (0, 0)),
            pl.BlockSpec(Wl1.shape, lambda b: (0, 0)),
            pl.BlockSpec((1, HC), lambda b: (0, 0)),
            pl.BlockSpec(Wr1.shape, lambda b: (0, 0)),
            pl.BlockSpec((1, HC), lambda b: (0, 0)),
            pl.BlockSpec((HC, H), lambda b: (0, 0)),
            pl.BlockSpec((1, HC), lambda b: (0, 0)),
            pl.BlockSpec(Wl2.shape, lambda b: (0, 0)),
            pl.BlockSpec((1, HC), lambda b: (0, 0)),
            pl.BlockSpec(Wr2.shape, lambda b: (0, 0)),
            pl.BlockSpec((1, HC), lambda b: (0, 0)),
            pl.BlockSpec((HC, H), lambda b: (0, 0)),
            pl.BlockSpec((1, HC), lambda b: (0, 0)),
            pl.BlockSpec(W_lin1.shape, lambda b: (0, 0)),
            pl.BlockSpec((1, 4 * HC), lambda b: (0, 0)),
            pl.BlockSpec(W_lin2.shape, lambda b: (0, 0)),
            pl.BlockSpec((1, HC), lambda b: (0, 0)),
            pl.BlockSpec(W_fin.shape, lambda b: (0, 0)),
            pl.BlockSpec((1, FT), lambda b: (0, 0)),
            pl.BlockSpec((H, HC), lambda b: (0, 0)),
        ],
        out_specs=[
            pl.BlockSpec((B, FT), lambda b: (0, 0)),
            pl.BlockSpec((B, HC), lambda b: (0, 0)),
        ],
    )

    out1, out2 = pl.pallas_call(
        _graph_head_kernel,
        grid_spec=grid_spec,
        out_shape=[
            jax.ShapeDtypeStruct((B, FT), f32),
            jax.ShapeDtypeStruct((B, HC), f32),
        ],
    )(
        hs3, pooled_output,
        W_p1, row(b_p1), W_p2, row(b_p2), row(ln_g), row(ln_b),
        Wl1, row(bl1), Wr1, row(br1), P1, row(bias1),
        Wl2, row(bl2), Wr2, row(br2), P2, row(bias2),
        W_lin1, row(b_lin1), W_lin2, row(b_lin2), W_fin, row(b_fin),
        Q,
    )
    return (out1, out2)
